# baseline (speedup 1.0000x reference)
import functools

import jax
import jax.numpy as jnp
from jax import lax
from jax.experimental import pallas as pl
from jax.experimental.pallas import tpu as pltpu
from jax.experimental.pallas import tpu_sc as plsc

T = 8
N = 10000
E = 160000
NHID = 128

NC = 2
NS = 16
NP = 10240
RPT = NP // NS
EPT = E // NS
CH = 128
NCH = 10240 // CH
EPADT = NCH * CH
PAD_DST = N + 200
NBUF = 2
NH = 2
NCHH = NCH // NH
RB = 1024
NRB = NP // RB

_mesh = lambda: plsc.VectorSubcoreMesh(core_axis_name="c", subcore_axis_name="s")
_DEG_SCATTER = True


def _scatter_body(grouped, ones_mode, table, src_idx, dst_idx, zeros, out,
                  acc, sbuf, dbuf, *rest):
    bufs = rest[:NBUF]
    gsems = rest[NBUF:2 * NBUF]
    ssems = rest[2 * NBUF:3 * NBUF]
    c = lax.axis_index("c")
    s = lax.axis_index("s")
    nt = T if grouped else T // NC

    def tview(ref, tt):
        t = tt if grouped else c * (T // NC) + tt
        return ref.at[c, t] if grouped else ref.at[t]

    def zero_own_rows():
        for k in range(RPT // CH):
            pltpu.sync_copy(bufs[1], acc.at[pl.ds(s * RPT + k * CH, CH)])

    pltpu.sync_copy(zeros.at[pl.ds(0, CH)], bufs[1])
    if ones_mode:
        pltpu.sync_copy(table, bufs[0])
    zero_own_rows()
    plsc.subcore_barrier()

    def per_t(tt, _):
        t = tt if grouped else c * (T // NC) + tt
        for hh in range(NH):
            hsl = pl.ds(hh * NCHH, NCHH)
            if not ones_mode:
                pltpu.sync_copy(src_idx.at[t, s].at[hsl], sbuf)
            pltpu.sync_copy(dst_idx.at[t, s].at[hsl], dbuf)

            if ones_mode:
                pltpu.async_copy(bufs[0], acc.at[dbuf.at[0]], ssems[0],
                                 add=True)

                def grp(g, _):
                    j0, j1 = 2 * g, 2 * g + 1
                    pltpu.async_copy(bufs[0], acc.at[dbuf.at[j1]], ssems[1],
                                     add=True)
                    pltpu.make_async_copy(
                        bufs[0], acc.at[dbuf.at[j0]], ssems[0]).wait()

                    @pl.when(j0 + 2 < NCHH)
                    def _():
                        pltpu.async_copy(bufs[0], acc.at[dbuf.at[j0 + 2]],
                                         ssems[0], add=True)

                    pltpu.make_async_copy(
                        bufs[0], acc.at[dbuf.at[j1]], ssems[1]).wait()
                    return 0

                lax.fori_loop(0, NCHH // 2, grp, 0)
            else:
                def gv(j):
                    return tview(table, tt).at[sbuf.at[j]]

                def sct(j, b, sem):
                    return pltpu.make_async_copy(bufs[b], acc.at[dbuf.at[j]],
                                                 sem)

                pltpu.async_copy(gv(0), bufs[0], gsems[0])

                def grp(g, _):
                    j0, j1 = 2 * g, 2 * g + 1
                    pltpu.make_async_copy(gv(j0), bufs[0], gsems[0]).wait()
                    pltpu.async_copy(bufs[0], acc.at[dbuf.at[j0]], ssems[0],
                                     add=True)

                    @pl.when(g > 0)
                    def _():
                        sct(j1 - 2, 1, ssems[1]).wait()

                    pltpu.async_copy(gv(j1), bufs[1], gsems[1])
                    pltpu.make_async_copy(gv(j1), bufs[1], gsems[1]).wait()
                    pltpu.async_copy(bufs[1], acc.at[dbuf.at[j1]], ssems[1],
                                     add=True)
                    sct(j0, 0, ssems[0]).wait()

                    @pl.when(j0 + 2 < NCHH)
                    def _():
                        pltpu.async_copy(gv(j0 + 2), bufs[0], gsems[0])
                    return 0

                lax.fori_loop(0, NCHH // 2, grp, 0)
                sct(NCHH - 1, 1, ssems[1]).wait()
        plsc.subcore_barrier()
        for k in range(RPT // CH):
            rows = pl.ds(s * RPT + k * CH, CH)
            pltpu.sync_copy(acc.at[rows], bufs[0])
            pltpu.sync_copy(bufs[0], tview(out, tt).at[rows])
        if ones_mode:
            pltpu.sync_copy(table, bufs[0])
        pltpu.sync_copy(zeros.at[pl.ds(0, CH)], bufs[1])
        zero_own_rows()
        plsc.subcore_barrier()
        return 0

    lax.fori_loop(0, nt, per_t, 0)


def _scatter(table, src_idx, dst_idx, zeros, grouped, ones_mode=False):
    shape = (NC, T, NP, NHID) if grouped else (T, NP, NHID)
    return pl.kernel(
        functools.partial(_scatter_body, grouped, ones_mode),
        out_type=jax.ShapeDtypeStruct(shape, jnp.float32),
        mesh=_mesh(),
        scratch_types=(
            [pltpu.VMEM_SHARED((NP, NHID), jnp.float32),
             pltpu.VMEM((NCHH, CH), jnp.int32),
             pltpu.VMEM((NCHH, CH), jnp.int32)]
            + [pltpu.VMEM((CH, NHID), jnp.float32)] * NBUF
            + [pltpu.SemaphoreType.DMA] * (2 * NBUF)
        ),
    )(table, src_idx, dst_idx, zeros)


def _tcA_body(emb_ref, w1_ref, deg_ref, y1_ref):
    xw = jnp.dot(emb_ref[...], w1_ref[...], preferred_element_type=jnp.float32)
    di = lax.rsqrt(deg_ref[0] + 1.0)
    y1_ref[0, 0] = xw[:, :NHID] * di
    y1_ref[1, 0] = xw[:, NHID:] * di


def _tcA(emb_pad, W1, deg_b):
    return pl.pallas_call(
        _tcA_body,
        grid=(T, NRB),
        in_specs=[
            pl.BlockSpec((RB, NHID), lambda t, r: (r, 0)),
            pl.BlockSpec((NHID, 2 * NHID), lambda t, r: (0, 0)),
            pl.BlockSpec((1, RB, NHID), lambda t, r: (t, r, 0)),
        ],
        out_specs=pl.BlockSpec((NC, 1, RB, NHID), lambda t, r: (0, t, r, 0)),
        out_shape=jax.ShapeDtypeStruct((NC, T, NP, NHID), jnp.float32),
    )(emb_pad, W1, deg_b)


def _tcB_body(z1_ref, y1_ref, deg_ref, w2_ref, b1_ref, y2_ref):
    di = lax.rsqrt(deg_ref[0] + 1.0)
    h_lo = (z1_ref[0, 0] + y1_ref[0, 0]) * di + b1_ref[0, :NHID]
    h_hi = (z1_ref[1, 0] + y1_ref[1, 0]) * di + b1_ref[0, NHID:]
    h = jnp.concatenate([h_lo, h_hi], axis=1)
    xw2 = jnp.dot(h, w2_ref[...], preferred_element_type=jnp.float32)
    y2_ref[0] = xw2 * di


def _tcB(z1, y1, deg_b, W2, b1):
    return pl.pallas_call(
        _tcB_body,
        grid=(T, NRB),
        in_specs=[
            pl.BlockSpec((NC, 1, RB, NHID), lambda t, r: (0, t, r, 0)),
            pl.BlockSpec((NC, 1, RB, NHID), lambda t, r: (0, t, r, 0)),
            pl.BlockSpec((1, RB, NHID), lambda t, r: (t, r, 0)),
            pl.BlockSpec((2 * NHID, NHID), lambda t, r: (0, 0)),
            pl.BlockSpec((1, 2 * NHID), lambda t, r: (0, 0)),
        ],
        out_specs=pl.BlockSpec((1, RB, NHID), lambda t, r: (t, r, 0)),
        out_shape=jax.ShapeDtypeStruct((T, NP, NHID), jnp.float32),
    )(z1, y1, deg_b, W2, b1.reshape(1, 2 * NHID))


def _tcC_body(z2_ref, y2_ref, deg_ref, b2_ref, out_ref):
    di = lax.rsqrt(deg_ref[0] + 1.0)
    out_ref[0] = (z2_ref[0] + y2_ref[0]) * di + b2_ref[0]


def _tcC(z2, y2, deg_b, b2):
    return pl.pallas_call(
        _tcC_body,
        grid=(T, NRB),
        in_specs=[
            pl.BlockSpec((1, RB, NHID), lambda t, r: (t, r, 0)),
            pl.BlockSpec((1, RB, NHID), lambda t, r: (t, r, 0)),
            pl.BlockSpec((1, RB, NHID), lambda t, r: (t, r, 0)),
            pl.BlockSpec((1, NHID), lambda t, r: (0, 0)),
        ],
        out_specs=pl.BlockSpec((1, RB, NHID), lambda t, r: (t, r, 0)),
        out_shape=jax.ShapeDtypeStruct((T, NP, NHID), jnp.float32),
    )(z2, y2, deg_b, b2.reshape(1, NHID))


def kernel(edge_index, emb, W1, b1, W2, b2):
    src = edge_index[:, 0, :].reshape(T, NS, EPT)
    dst = edge_index[:, 1, :].reshape(T, NS, EPT)
    pad = EPADT - EPT
    src_idx = jnp.pad(src, ((0, 0), (0, 0), (0, pad))).reshape(T, NS, NCH, CH)
    dst_idx = jnp.pad(dst, ((0, 0), (0, 0), (0, pad)),
                      constant_values=PAD_DST).reshape(T, NS, NCH, CH)
    emb_pad = jnp.pad(emb, ((0, NP - N), (0, 0)))
    zeros = jnp.zeros((NP, NHID), jnp.float32)
    ones_rows = jnp.ones((CH, NHID), jnp.float32)

    deg_b = _scatter(ones_rows, src_idx, dst_idx, zeros, False, True)

    y1 = _tcA(emb_pad, W1, deg_b)
    z1 = _scatter(y1, src_idx, dst_idx, zeros, True)

    y2 = _tcB(z1, y1, deg_b, W2, b1)
    z2 = _scatter(y2, src_idx, dst_idx, zeros, False)

    out = _tcC(z2, y2, deg_b, b2)
    return out[:, :N, :]


# speedup vs baseline: 1.0078x; 1.0078x over previous
import functools

import jax
import jax.numpy as jnp
from jax import lax
from jax.experimental import pallas as pl
from jax.experimental.pallas import tpu as pltpu
from jax.experimental.pallas import tpu_sc as plsc

T = 8
N = 10000
E = 160000
NHID = 128

NC = 2
NS = 16
NP = 10240
RPT = NP // NS
EPT = E // NS
CH = 128
NCH = 10240 // CH
EPADT = NCH * CH
PAD_DST = N + 200
NBUF = 2
NH = 2
NCHH = NCH // NH
RB = 1024
NRB = NP // RB

_mesh = lambda: plsc.VectorSubcoreMesh(core_axis_name="c", subcore_axis_name="s")


def _scatter_body(grouped, ones_mode, table, src_idx, dst_idx, zeros, out,
                  acc, sbuf, dbuf, *rest):
    bufs = rest[:NBUF]
    gsems = rest[NBUF:2 * NBUF]
    ssems = rest[2 * NBUF:3 * NBUF]
    c = lax.axis_index("c")
    s = lax.axis_index("s")
    nt = T if grouped else T // NC

    def tview(ref, tt):
        t = tt if grouped else c * (T // NC) + tt
        return ref.at[c, t] if grouped else ref.at[t]

    def zero_own_rows():
        for k in range(RPT // CH):
            pltpu.sync_copy(bufs[1], acc.at[pl.ds(s * RPT + k * CH, CH)])

    pltpu.sync_copy(zeros.at[pl.ds(0, CH)], bufs[1])
    if ones_mode:
        pltpu.sync_copy(table, bufs[0])
    zero_own_rows()
    plsc.subcore_barrier()

    def per_t(tt, _):
        t = tt if grouped else c * (T // NC) + tt
        for hh in range(NH):
            hsl = pl.ds(hh * NCHH, NCHH)
            if not ones_mode:
                pltpu.sync_copy(src_idx.at[t, s].at[hsl], sbuf)
            pltpu.sync_copy(dst_idx.at[t, s].at[hsl], dbuf)

            if ones_mode:
                pltpu.async_copy(bufs[0], acc.at[dbuf.at[0]], ssems[0],
                                 add=True)

                def grp(g, _):
                    j0, j1 = 2 * g, 2 * g + 1
                    pltpu.async_copy(bufs[0], acc.at[dbuf.at[j1]], ssems[1],
                                     add=True)
                    pltpu.make_async_copy(
                        bufs[0], acc.at[dbuf.at[j0]], ssems[0]).wait()

                    @pl.when(j0 + 2 < NCHH)
                    def _():
                        pltpu.async_copy(bufs[0], acc.at[dbuf.at[j0 + 2]],
                                         ssems[0], add=True)

                    pltpu.make_async_copy(
                        bufs[0], acc.at[dbuf.at[j1]], ssems[1]).wait()
                    return 0

                lax.fori_loop(0, NCHH // 2, grp, 0)
            else:
                def gv(j):
                    return tview(table, tt).at[sbuf.at[j]]

                def sct(j, b, sem):
                    return pltpu.make_async_copy(bufs[b], acc.at[dbuf.at[j]],
                                                 sem)

                pltpu.async_copy(gv(0), bufs[0], gsems[0])

                def grp(g, _):
                    j0, j1 = 2 * g, 2 * g + 1
                    pltpu.make_async_copy(gv(j0), bufs[0], gsems[0]).wait()
                    pltpu.async_copy(bufs[0], acc.at[dbuf.at[j0]], ssems[0],
                                     add=True)

                    @pl.when(g > 0)
                    def _():
                        sct(j1 - 2, 1, ssems[1]).wait()

                    pltpu.async_copy(gv(j1), bufs[1], gsems[1])
                    pltpu.make_async_copy(gv(j1), bufs[1], gsems[1]).wait()
                    pltpu.async_copy(bufs[1], acc.at[dbuf.at[j1]], ssems[1],
                                     add=True)
                    sct(j0, 0, ssems[0]).wait()

                    @pl.when(j0 + 2 < NCHH)
                    def _():
                        pltpu.async_copy(gv(j0 + 2), bufs[0], gsems[0])
                    return 0

                lax.fori_loop(0, NCHH // 2, grp, 0)
                sct(NCHH - 1, 1, ssems[1]).wait()
        plsc.subcore_barrier()
        for k in range(RPT // CH):
            rows = pl.ds(s * RPT + k * CH, CH)
            pltpu.sync_copy(acc.at[rows], bufs[0])
            pltpu.sync_copy(bufs[0], tview(out, tt).at[rows])
        if ones_mode:
            pltpu.sync_copy(table, bufs[0])
        pltpu.sync_copy(zeros.at[pl.ds(0, CH)], bufs[1])
        zero_own_rows()
        plsc.subcore_barrier()
        return 0

    lax.fori_loop(0, nt, per_t, 0)


def _scatter(table, src_idx, dst_idx, zeros, grouped, ones_mode=False):
    shape = (NC, T, NP, NHID) if grouped else (T, NP, NHID)
    dt = table.dtype
    return pl.kernel(
        functools.partial(_scatter_body, grouped, ones_mode),
        out_type=jax.ShapeDtypeStruct(shape, dt),
        mesh=_mesh(),
        scratch_types=(
            [pltpu.VMEM_SHARED((NP, NHID), dt),
             pltpu.VMEM((NCHH, CH), jnp.int32),
             pltpu.VMEM((NCHH, CH), jnp.int32)]
            + [pltpu.VMEM((CH, NHID), dt)] * NBUF
            + [pltpu.SemaphoreType.DMA] * (2 * NBUF)
        ),
    )(table, src_idx, dst_idx, zeros)


def _tcA_body(emb_ref, w1_ref, deg_ref, y1_ref):
    xw = jnp.dot(emb_ref[...], w1_ref[...], preferred_element_type=jnp.float32)
    di = lax.rsqrt(deg_ref[0] + 1.0)
    y1_ref[0, 0] = xw[:, :NHID] * di
    y1_ref[1, 0] = xw[:, NHID:] * di


def _tcA(emb_pad, W1, deg_b):
    return pl.pallas_call(
        _tcA_body,
        grid=(T, NRB),
        in_specs=[
            pl.BlockSpec((RB, NHID), lambda t, r: (r, 0)),
            pl.BlockSpec((NHID, 2 * NHID), lambda t, r: (0, 0)),
            pl.BlockSpec((1, RB, NHID), lambda t, r: (t, r, 0)),
        ],
        out_specs=pl.BlockSpec((NC, 1, RB, NHID), lambda t, r: (0, t, r, 0)),
        out_shape=jax.ShapeDtypeStruct((NC, T, NP, NHID), jnp.float32),
    )(emb_pad, W1, deg_b)


def _tcB_body(z1_ref, y1_ref, deg_ref, w2_ref, b1_ref, y2_ref):
    di = lax.rsqrt(deg_ref[0] + 1.0)
    h_lo = (z1_ref[0, 0] + y1_ref[0, 0]) * di + b1_ref[0, :NHID]
    h_hi = (z1_ref[1, 0] + y1_ref[1, 0]) * di + b1_ref[0, NHID:]
    h = jnp.concatenate([h_lo, h_hi], axis=1)
    xw2 = jnp.dot(h, w2_ref[...], preferred_element_type=jnp.float32)
    y2_ref[0] = xw2 * di


def _tcB(z1, y1, deg_b, W2, b1):
    return pl.pallas_call(
        _tcB_body,
        grid=(T, NRB),
        in_specs=[
            pl.BlockSpec((NC, 1, RB, NHID), lambda t, r: (0, t, r, 0)),
            pl.BlockSpec((NC, 1, RB, NHID), lambda t, r: (0, t, r, 0)),
            pl.BlockSpec((1, RB, NHID), lambda t, r: (t, r, 0)),
            pl.BlockSpec((2 * NHID, NHID), lambda t, r: (0, 0)),
            pl.BlockSpec((1, 2 * NHID), lambda t, r: (0, 0)),
        ],
        out_specs=pl.BlockSpec((1, RB, NHID), lambda t, r: (t, r, 0)),
        out_shape=jax.ShapeDtypeStruct((T, NP, NHID), jnp.float32),
    )(z1, y1, deg_b, W2, b1.reshape(1, 2 * NHID))


def _tcC_body(z2_ref, y2_ref, deg_ref, b2_ref, out_ref):
    di = lax.rsqrt(deg_ref[0] + 1.0)
    out_ref[0] = (z2_ref[0] + y2_ref[0]) * di + b2_ref[0]


def _tcC(z2, y2, deg_b, b2):
    return pl.pallas_call(
        _tcC_body,
        grid=(T, NRB),
        in_specs=[
            pl.BlockSpec((1, RB, NHID), lambda t, r: (t, r, 0)),
            pl.BlockSpec((1, RB, NHID), lambda t, r: (t, r, 0)),
            pl.BlockSpec((1, RB, NHID), lambda t, r: (t, r, 0)),
            pl.BlockSpec((1, NHID), lambda t, r: (0, 0)),
        ],
        out_specs=pl.BlockSpec((1, RB, NHID), lambda t, r: (t, r, 0)),
        out_shape=jax.ShapeDtypeStruct((T, NP, NHID), jnp.float32),
    )(z2, y2, deg_b, b2.reshape(1, NHID))


def kernel(edge_index, emb, W1, b1, W2, b2):
    src = edge_index[:, 0, :].reshape(T, NS, EPT)
    dst = edge_index[:, 1, :].reshape(T, NS, EPT)
    pad = EPADT - EPT
    src_idx = jnp.pad(src, ((0, 0), (0, 0), (0, pad))).reshape(T, NS, NCH, CH)
    dst_idx = jnp.pad(dst, ((0, 0), (0, 0), (0, pad)),
                      constant_values=PAD_DST).reshape(T, NS, NCH, CH)
    emb_pad = jnp.pad(emb, ((0, NP - N), (0, 0)))
    zeros = jnp.zeros((NP, NHID), jnp.float32)
    ones_rows = jnp.ones((CH, NHID), jnp.float32)

    deg_b = _scatter(ones_rows, src_idx, dst_idx, zeros, False, True)

    y1 = _tcA(emb_pad, W1, deg_b)
    z1 = _scatter(y1, src_idx, dst_idx, zeros, True)

    y2 = _tcB(z1, y1, deg_b, W2, b1)
    z2 = _scatter(y2, src_idx, dst_idx, zeros, False)

    out = _tcC(z2, y2, deg_b, b2)
    return out[:, :N, :]


# speedup vs baseline: 1.0101x; 1.0022x over previous
import functools

import jax
import jax.numpy as jnp
from jax import lax
from jax.experimental import pallas as pl
from jax.experimental.pallas import tpu as pltpu
from jax.experimental.pallas import tpu_sc as plsc

T = 8
N = 10000
E = 160000
NHID = 128

NC = 2
NS = 16
NP = 10240
RPT = NP // NS
EPT = E // NS
CH = 128
NCH = 10240 // CH
EPADT = NCH * CH
PAD_DST = N + 200
NBUF = 2
NH = 2
NCHH = NCH // NH
RB = 1024
NRB = NP // RB

_mesh = lambda: plsc.VectorSubcoreMesh(core_axis_name="c", subcore_axis_name="s")


def _deg16_body(dst_idx, deg_out, acc, dbuf, ones_v, z16, stg, *ssems):
    c = lax.axis_index("c")
    s = lax.axis_index("s")
    for i in range(CH):
        ones_v[i, :] = jnp.ones((16,), jnp.float32)
    for i in range(40):
        z16[i, :] = jnp.zeros((16,), jnp.float32)

    def zero_own():
        for k in range(RPT // 40):
            pltpu.sync_copy(z16, acc.at[pl.ds(s * RPT + k * 40, 40)])

    zero_own()
    plsc.subcore_barrier()
    for tt in range(T // NC):
        t = c * (T // NC) + tt
        pltpu.sync_copy(dst_idx.at[t, s], dbuf)
        pltpu.async_copy(ones_v, acc.at[dbuf.at[0]], ssems[0], add=True)

        def grp(g, _):
            j0, j1 = 2 * g, 2 * g + 1
            pltpu.async_copy(ones_v, acc.at[dbuf.at[j1]], ssems[1], add=True)
            pltpu.make_async_copy(ones_v, acc.at[dbuf.at[j0]], ssems[0]).wait()

            @pl.when(j0 + 2 < NCH)
            def _():
                pltpu.async_copy(ones_v, acc.at[dbuf.at[j0 + 2]], ssems[0],
                                 add=True)

            pltpu.make_async_copy(ones_v, acc.at[dbuf.at[j1]], ssems[1]).wait()
            return 0

        lax.fori_loop(0, NCH // 2, grp, 0)
        plsc.subcore_barrier()
        for k in range(RPT // 64):
            rows = pl.ds(s * RPT + k * 64, 64)
            pltpu.sync_copy(acc.at[rows], stg)
            pltpu.sync_copy(stg, deg_out.at[t].at[rows])
        zero_own()
        plsc.subcore_barrier()


def _deg16(dst_idx):
    return pl.kernel(
        _deg16_body,
        out_type=jax.ShapeDtypeStruct((T, NP, 16), jnp.float32),
        mesh=_mesh(),
        scratch_types=[
            pltpu.VMEM_SHARED((NP, 16), jnp.float32),
            pltpu.VMEM((NCH, CH), jnp.int32),
            pltpu.VMEM((CH, 16), jnp.float32),
            pltpu.VMEM((40, 16), jnp.float32),
            pltpu.VMEM((64, 16), jnp.float32),
            pltpu.SemaphoreType.DMA,
            pltpu.SemaphoreType.DMA,
        ],
    )(dst_idx)


def _scatter_body(grouped, ones_mode, table, src_idx, dst_idx, zeros, out,
                  acc, sbuf, dbuf, *rest):
    bufs = rest[:NBUF]
    gsems = rest[NBUF:2 * NBUF]
    ssems = rest[2 * NBUF:3 * NBUF]
    c = lax.axis_index("c")
    s = lax.axis_index("s")
    nt = T if grouped else T // NC

    def tview(ref, tt):
        t = tt if grouped else c * (T // NC) + tt
        return ref.at[c, t] if grouped else ref.at[t]

    def zero_own_rows():
        for k in range(RPT // CH):
            pltpu.sync_copy(bufs[1], acc.at[pl.ds(s * RPT + k * CH, CH)])

    pltpu.sync_copy(zeros.at[pl.ds(0, CH)], bufs[1])
    if ones_mode:
        pltpu.sync_copy(table, bufs[0])
    zero_own_rows()
    plsc.subcore_barrier()

    def per_t(tt, _):
        t = tt if grouped else c * (T // NC) + tt
        for hh in range(NH):
            hsl = pl.ds(hh * NCHH, NCHH)
            if not ones_mode:
                pltpu.sync_copy(src_idx.at[t, s].at[hsl], sbuf)
            pltpu.sync_copy(dst_idx.at[t, s].at[hsl], dbuf)

            if ones_mode:
                pltpu.async_copy(bufs[0], acc.at[dbuf.at[0]], ssems[0],
                                 add=True)

                def grp(g, _):
                    j0, j1 = 2 * g, 2 * g + 1
                    pltpu.async_copy(bufs[0], acc.at[dbuf.at[j1]], ssems[1],
                                     add=True)
                    pltpu.make_async_copy(
                        bufs[0], acc.at[dbuf.at[j0]], ssems[0]).wait()

                    @pl.when(j0 + 2 < NCHH)
                    def _():
                        pltpu.async_copy(bufs[0], acc.at[dbuf.at[j0 + 2]],
                                         ssems[0], add=True)

                    pltpu.make_async_copy(
                        bufs[0], acc.at[dbuf.at[j1]], ssems[1]).wait()
                    return 0

                lax.fori_loop(0, NCHH // 2, grp, 0)
            else:
                def gv(j):
                    return tview(table, tt).at[sbuf.at[j]]

                def sct(j, b, sem):
                    return pltpu.make_async_copy(bufs[b], acc.at[dbuf.at[j]],
                                                 sem)

                pltpu.async_copy(gv(0), bufs[0], gsems[0])

                def grp(g, _):
                    j0, j1 = 2 * g, 2 * g + 1
                    pltpu.make_async_copy(gv(j0), bufs[0], gsems[0]).wait()
                    pltpu.async_copy(bufs[0], acc.at[dbuf.at[j0]], ssems[0],
                                     add=True)

                    @pl.when(g > 0)
                    def _():
                        sct(j1 - 2, 1, ssems[1]).wait()

                    pltpu.async_copy(gv(j1), bufs[1], gsems[1])
                    pltpu.make_async_copy(gv(j1), bufs[1], gsems[1]).wait()
                    pltpu.async_copy(bufs[1], acc.at[dbuf.at[j1]], ssems[1],
                                     add=True)
                    sct(j0, 0, ssems[0]).wait()

                    @pl.when(j0 + 2 < NCHH)
                    def _():
                        pltpu.async_copy(gv(j0 + 2), bufs[0], gsems[0])
                    return 0

                lax.fori_loop(0, NCHH // 2, grp, 0)
                sct(NCHH - 1, 1, ssems[1]).wait()
        plsc.subcore_barrier()
        for k in range(RPT // CH):
            rows = pl.ds(s * RPT + k * CH, CH)
            pltpu.sync_copy(acc.at[rows], bufs[0])
            pltpu.sync_copy(bufs[0], tview(out, tt).at[rows])
        if ones_mode:
            pltpu.sync_copy(table, bufs[0])
        pltpu.sync_copy(zeros.at[pl.ds(0, CH)], bufs[1])
        zero_own_rows()
        plsc.subcore_barrier()
        return 0

    lax.fori_loop(0, nt, per_t, 0)


def _scatter(table, src_idx, dst_idx, zeros, grouped, ones_mode=False):
    shape = (NC, T, NP, NHID) if grouped else (T, NP, NHID)
    dt = table.dtype
    return pl.kernel(
        functools.partial(_scatter_body, grouped, ones_mode),
        out_type=jax.ShapeDtypeStruct(shape, dt),
        mesh=_mesh(),
        scratch_types=(
            [pltpu.VMEM_SHARED((NP, NHID), dt),
             pltpu.VMEM((NCHH, CH), jnp.int32),
             pltpu.VMEM((NCHH, CH), jnp.int32)]
            + [pltpu.VMEM((CH, NHID), dt)] * NBUF
            + [pltpu.SemaphoreType.DMA] * (2 * NBUF)
        ),
    )(table, src_idx, dst_idx, zeros)


def _tcA_body(emb_ref, w1_ref, deg_ref, y1_ref):
    xw = jnp.dot(emb_ref[...], w1_ref[...], preferred_element_type=jnp.float32)
    di = lax.rsqrt(deg_ref[0][:, 0:1] + 1.0)
    y1_ref[0, 0] = xw[:, :NHID] * di
    y1_ref[1, 0] = xw[:, NHID:] * di


def _tcA(emb_pad, W1, deg_b):
    return pl.pallas_call(
        _tcA_body,
        grid=(T, NRB),
        in_specs=[
            pl.BlockSpec((RB, NHID), lambda t, r: (r, 0)),
            pl.BlockSpec((NHID, 2 * NHID), lambda t, r: (0, 0)),
            pl.BlockSpec((1, RB, 16), lambda t, r: (t, r, 0)),
        ],
        out_specs=pl.BlockSpec((NC, 1, RB, NHID), lambda t, r: (0, t, r, 0)),
        out_shape=jax.ShapeDtypeStruct((NC, T, NP, NHID), jnp.float32),
    )(emb_pad, W1, deg_b)


def _tcB_body(z1_ref, y1_ref, deg_ref, w2_ref, b1_ref, y2_ref):
    di = lax.rsqrt(deg_ref[0][:, 0:1] + 1.0)
    h_lo = (z1_ref[0, 0] + y1_ref[0, 0]) * di + b1_ref[0, :NHID]
    h_hi = (z1_ref[1, 0] + y1_ref[1, 0]) * di + b1_ref[0, NHID:]
    h = jnp.concatenate([h_lo, h_hi], axis=1)
    xw2 = jnp.dot(h, w2_ref[...], preferred_element_type=jnp.float32)
    y2_ref[0] = xw2 * di


def _tcB(z1, y1, deg_b, W2, b1):
    return pl.pallas_call(
        _tcB_body,
        grid=(T, NRB),
        in_specs=[
            pl.BlockSpec((NC, 1, RB, NHID), lambda t, r: (0, t, r, 0)),
            pl.BlockSpec((NC, 1, RB, NHID), lambda t, r: (0, t, r, 0)),
            pl.BlockSpec((1, RB, 16), lambda t, r: (t, r, 0)),
            pl.BlockSpec((2 * NHID, NHID), lambda t, r: (0, 0)),
            pl.BlockSpec((1, 2 * NHID), lambda t, r: (0, 0)),
        ],
        out_specs=pl.BlockSpec((1, RB, NHID), lambda t, r: (t, r, 0)),
        out_shape=jax.ShapeDtypeStruct((T, NP, NHID), jnp.float32),
    )(z1, y1, deg_b, W2, b1.reshape(1, 2 * NHID))


def _tcC_body(z2_ref, y2_ref, deg_ref, b2_ref, out_ref):
    di = lax.rsqrt(deg_ref[0][:, 0:1] + 1.0)
    out_ref[0] = (z2_ref[0] + y2_ref[0]) * di + b2_ref[0]


def _tcC(z2, y2, deg_b, b2):
    return pl.pallas_call(
        _tcC_body,
        grid=(T, NRB),
        in_specs=[
            pl.BlockSpec((1, RB, NHID), lambda t, r: (t, r, 0)),
            pl.BlockSpec((1, RB, NHID), lambda t, r: (t, r, 0)),
            pl.BlockSpec((1, RB, 16), lambda t, r: (t, r, 0)),
            pl.BlockSpec((1, NHID), lambda t, r: (0, 0)),
        ],
        out_specs=pl.BlockSpec((1, RB, NHID), lambda t, r: (t, r, 0)),
        out_shape=jax.ShapeDtypeStruct((T, NP, NHID), jnp.float32),
    )(z2, y2, deg_b, b2.reshape(1, NHID))


def kernel(edge_index, emb, W1, b1, W2, b2):
    src = edge_index[:, 0, :].reshape(T, NS, EPT)
    dst = edge_index[:, 1, :].reshape(T, NS, EPT)
    pad = EPADT - EPT
    src_idx = jnp.pad(src, ((0, 0), (0, 0), (0, pad))).reshape(T, NS, NCH, CH)
    dst_idx = jnp.pad(dst, ((0, 0), (0, 0), (0, pad)),
                      constant_values=PAD_DST).reshape(T, NS, NCH, CH)
    emb_pad = jnp.pad(emb, ((0, NP - N), (0, 0)))
    zeros = jnp.zeros((NP, NHID), jnp.float32)
    ones_rows = jnp.ones((CH, NHID), jnp.float32)

    deg_b = _deg16(dst_idx)
    del ones_rows

    y1 = _tcA(emb_pad, W1, deg_b)
    z1 = _scatter(y1, src_idx, dst_idx, zeros, True)

    y2 = _tcB(z1, y1, deg_b, W2, b1)
    z2 = _scatter(y2, src_idx, dst_idx, zeros, False)

    out = _tcC(z2, y2, deg_b, b2)
    return out[:, :N, :]
